# baseline (speedup 1.0000x reference)
import functools

import jax
import jax.numpy as jnp
from jax import lax
from jax.experimental import pallas as pl
from jax.experimental.pallas import tpu as pltpu
from jax.experimental.pallas import tpu_sc as plsc

_T = 100000
_V = 50000
_NW = 32
_C = 1024
_GPC = _C // 16
_CHUNKS = 3
_MAIN_T = _NW * _CHUNKS * _C
_TAIL_GROUPS = (_T - _MAIN_T) // 16
_MAIN_G = _MAIN_T // 16


def _compute_group(verts_v, bary_v, out_v, t0):
    iota = lax.iota(jnp.int32, 16)
    r3 = 3 * t0 + 3 * iota
    fb = 18 * t0 + 18 * iota
    v = [[plsc.load_gather(verts_v, [r3 + k, jnp.full((16,), j, jnp.int32)])
          for j in range(3)] for k in range(3)]
    for p in range(6):
        b = [plsc.load_gather(bary_v, [fb + (3 * p + i)]) for i in range(3)]
        r = 1.0 / (b[0] + b[1] + b[2])
        for j in range(3):
            acc = b[0] * v[0][j] + b[1] * v[1][j] + b[2] * v[2][j]
            plsc.store_scatter(out_v, [fb + (3 * p + j)], acc * r)


def _mesh_body(tri_hbm, bary_hbm, table_hbm, out_hbm,
               idx_v, verts_v, bary_v, out_v, sem):
    wid = lax.axis_index("s") * 2 + lax.axis_index("c")

    def chunk_body(c, carry):
        t_base = (wid * _CHUNKS + c) * _C
        pltpu.sync_copy(tri_hbm.at[pl.ds(3 * t_base, 3 * _C)], idx_v)
        copies = [
            pltpu.async_copy(table_hbm.at[idx_v.at[pl.ds(128 * j, 128)]],
                             verts_v.at[pl.ds(128 * j, 128)], sem)
            for j in range(3 * _C // 128)
        ]
        for cp in copies:
            cp.wait()
        pltpu.sync_copy(bary_hbm.at[pl.ds(18 * t_base, 18 * _C)], bary_v)

        def group_body(g, carry2):
            _compute_group(verts_v, bary_v, out_v, 16 * g)
            return carry2

        lax.fori_loop(0, _GPC, group_body, 0)
        pltpu.sync_copy(out_v, out_hbm.at[pl.ds(18 * t_base, 18 * _C)])
        return carry

    lax.fori_loop(0, _CHUNKS, chunk_body, 0)

    n_tail = jnp.where(wid < _TAIL_GROUPS - 3 * _NW, 4, 3)

    def tail_body(k, carry):
        t_base = 16 * (_MAIN_G + wid + _NW * k)
        pltpu.sync_copy(tri_hbm.at[pl.ds(3 * t_base, 48)],
                        idx_v.at[pl.ds(0, 48)])
        pltpu.async_copy(table_hbm.at[idx_v.at[pl.ds(0, 48)]],
                         verts_v.at[pl.ds(0, 48)], sem).wait()
        pltpu.sync_copy(bary_hbm.at[pl.ds(18 * t_base, 288)],
                        bary_v.at[pl.ds(0, 288)])
        _compute_group(verts_v, bary_v, out_v, 0)
        pltpu.sync_copy(out_v.at[pl.ds(0, 288)],
                        out_hbm.at[pl.ds(18 * t_base, 288)])
        return carry

    lax.fori_loop(0, n_tail, tail_body, 0)


_mesh_kernel = pl.kernel(
    _mesh_body,
    out_type=jax.ShapeDtypeStruct((18 * _T,), jnp.float32),
    mesh=plsc.VectorSubcoreMesh(core_axis_name="c", subcore_axis_name="s"),
    scratch_types=[
        pltpu.VMEM((3 * _C,), jnp.int32),
        pltpu.VMEM((3 * _C, 8), jnp.float32),
        pltpu.VMEM((18 * _C,), jnp.float32),
        pltpu.VMEM((18 * _C,), jnp.float32),
        pltpu.SemaphoreType.DMA,
    ],
    compiler_params=pltpu.CompilerParams(
        needs_layout_passes=False, use_tc_tiling_on_sc=False),
)


def kernel(vertex_coords, bary_coords, triangles):
    tri_flat = triangles.reshape(-1)
    bary_flat = bary_coords.reshape(-1)
    table8 = jnp.pad(vertex_coords, ((0, 0), (0, 5)))
    out_flat = _mesh_kernel(tri_flat, bary_flat, table8)
    return out_flat.reshape(_T * 6, 3)
